# Initial kernel scaffold; baseline (speedup 1.0000x reference)
import jax
import jax.numpy as jnp
from jax.experimental import pallas as pl


def kernel(x, edge_index, seq_reverse, seqid_reverse, row, ln_g, ln_b, llin_W, llin_b, rlin_W, rlin_b, gru_Wih, gru_Whh, gru_bih, gru_bhh, sage_Wl, sage_bl, sage_Wr, lin_W, lin_b):
    raise NotImplementedError("write your pallas kernel here")


# speedup vs baseline: 3.2245x; 3.2245x over previous
import functools

import jax
import jax.numpy as jnp
from jax import lax
from jax.experimental import pallas as pl
from jax.experimental.pallas import tpu as pltpu
from jax.experimental.pallas import tpu_sc as plsc

N, E, F, H, C, L = 10000, 160000, 256, 256, 40, 1024
FH = F // 2
NSUB = 16
BLK = 80
NBLK = E // NSUB // BLK
ROWS_A = 640
ROWS_TAIL = N - (NSUB - 1) * ROWS_A
DEG0 = 63


@functools.cache
def _make_sc_segsum():
    mesh = plsc.VectorSubcoreMesh(core_axis_name="c", subcore_axis_name="s")
    return functools.partial(
        pl.kernel,
        out_type=(
            jax.ShapeDtypeStruct((2, N, FH), jnp.float32),
            jax.ShapeDtypeStruct((2, N, FH), jnp.float32),
        ),
        mesh=mesh,
        scratch_types=[
            pltpu.VMEM((BLK,), jnp.int32),
            pltpu.VMEM((BLK,), jnp.int32),
            pltpu.VMEM((BLK, FH), jnp.float32),
            pltpu.VMEM((BLK, FH), jnp.float32),
            pltpu.VMEM_SHARED((N, FH), jnp.float32),
            pltpu.SemaphoreType.DMA,
        ],
    )(_sc_segsum_body)


def _sc_segsum_body(xcat_hbm, src_hbm, dst_hbm, zf_hbm,
                    agg_hbm, deg_hbm,
                    si_v, di_v, rows_v, ones_v, acc_sh, sem):
    c = lax.axis_index("c")
    s = lax.axis_index("s")
    row0 = pl.multiple_of(s * ROWS_A, 8)

    def zero_my_slice():
        @pl.when(s < NSUB - 1)
        def _():
            pltpu.sync_copy(zf_hbm.at[pl.ds(row0, ROWS_A)],
                            acc_sh.at[pl.ds(row0, ROWS_A)])

        @pl.when(s == NSUB - 1)
        def _():
            pltpu.sync_copy(zf_hbm.at[pl.ds(N - ROWS_TAIL, ROWS_TAIL)],
                            acc_sh.at[pl.ds(N - ROWS_TAIL, ROWS_TAIL)])

    def copy_out_my_slice(out):
        @pl.when(s < NSUB - 1)
        def _():
            pltpu.sync_copy(acc_sh.at[pl.ds(row0, ROWS_A)],
                            out.at[c, pl.ds(row0, ROWS_A)])

        @pl.when(s == NSUB - 1)
        def _():
            pltpu.sync_copy(acc_sh.at[pl.ds(N - ROWS_TAIL, ROWS_TAIL)],
                            out.at[c, pl.ds(N - ROWS_TAIL, ROWS_TAIL)])

    zero_my_slice()

    def _init_ones(i, carry):
        def _in2(j, carry2):
            ones_v[i, pl.ds(j * 16, 16)] = jnp.ones((16,), jnp.float32)
            return carry2
        return lax.fori_loop(0, FH // 16, _in2, carry)
    lax.fori_loop(0, BLK, _init_ones, 0)

    plsc.subcore_barrier()
    e0 = s * (E // NSUB)
    coff = c * N

    def _blk(b, carry):
        pltpu.sync_copy(src_hbm.at[pl.ds(e0 + b * BLK, BLK)], si_v)
        pltpu.sync_copy(dst_hbm.at[pl.ds(e0 + b * BLK, BLK)], di_v)

        def _off(kk, carry2):
            si_v[pl.ds(kk * 16, 16)] = (si_v[pl.ds(kk * 16, 16)]
                                        + coff * jnp.ones((16,), jnp.int32))
            return carry2
        lax.fori_loop(0, BLK // 16, _off, 0)

        pltpu.async_copy(xcat_hbm.at[si_v], rows_v, sem).wait()
        pltpu.sync_copy(rows_v, acc_sh.at[di_v], add=True)
        return carry

    lax.fori_loop(0, NBLK, _blk, 0)
    plsc.subcore_barrier()
    copy_out_my_slice(agg_hbm)
    zero_my_slice()
    plsc.subcore_barrier()

    def _dblk(b, carry):
        @pl.when(b < DEG0 - c)
        def _():
            be = b + DEG0 * c
            pltpu.sync_copy(dst_hbm.at[pl.ds(e0 + be * BLK, BLK)], di_v)
            pltpu.sync_copy(ones_v, acc_sh.at[di_v], add=True)
        return carry

    lax.fori_loop(0, DEG0, _dblk, 0)
    plsc.subcore_barrier()
    copy_out_my_slice(deg_hbm)


def _seq_body(seq_ref, lng_ref, lnb_ref, lw_ref, lb_ref, rw_ref, rb_ref,
              wih_ref, whh_ref, bih_ref, bhh_ref, out_ref, gi_ref, l_ref):
    sv = seq_ref[...]
    mu = jnp.mean(sv, axis=-1, keepdims=True)
    var = jnp.mean((sv - mu) * (sv - mu), axis=-1, keepdims=True)
    sn = (sv - mu) / jnp.sqrt(var + 1e-5) * lng_ref[...] + lnb_ref[...]

    dn = (((1,), (1,)), ((), ()))
    lin = lax.dot_general(sn, lw_ref[...], dn,
                          preferred_element_type=jnp.float32) + lb_ref[...]
    l_ref[...] = lin * 0.5 * (1.0 + lax.erf(lin * 0.7071067811865475))
    rin = lax.dot_general(sn, rw_ref[...], dn,
                          preferred_element_type=jnp.float32) + rb_ref[...]
    gi_ref[...] = lax.dot_general(rin, wih_ref[...], dn,
                                  preferred_element_type=jnp.float32) + bih_ref[...]

    whh = whh_ref[...]
    bhh = bhh_ref[...]

    def _step(t, h):
        gi = gi_ref[pl.ds(t, 1), :]
        gh = lax.dot_general(h, whh, dn,
                             preferred_element_type=jnp.float32) + bhh
        r = jax.nn.sigmoid(gi[:, 0:H] + gh[:, 0:H])
        z = jax.nn.sigmoid(gi[:, H:2 * H] + gh[:, H:2 * H])
        n = jnp.tanh(gi[:, 2 * H:] + r * gh[:, 2 * H:])
        hn = (1.0 - z) * n + z * h
        out_ref[pl.ds(t, 1), :] = l_ref[pl.ds(t, 1), :] * hn
        return hn

    lax.fori_loop(0, L, _step, jnp.zeros((1, H), jnp.float32))


BN = 400
NBN = N // BN


def _fin_body(x_ref, agg_ref, deg_ref, row_ref, seq_ref, wl_ref, bl_ref,
              wr_ref, lw_ref, lb_ref, out_ref):
    deg = deg_ref[0, :, 0:1] + deg_ref[1, :, 0:1]
    recip = 1.0 / jnp.maximum(deg, 1.0)
    m0 = agg_ref[0] * recip
    m1 = agg_ref[1] * recip
    dn = (((1,), (1,)), ((), ()))
    wl = wl_ref[...]
    h = (lax.dot_general(m0, wl[:, :FH], dn, preferred_element_type=jnp.float32)
         + lax.dot_general(m1, wl[:, FH:], dn, preferred_element_type=jnp.float32)
         + bl_ref[...]
         + lax.dot_general(x_ref[...], wr_ref[...], dn,
                           preferred_element_type=jnp.float32))
    h = jnp.maximum(h, 0.0)

    rowv = row_ref[0, 0, :]
    onehot = (lax.broadcasted_iota(jnp.int32, (BN, L), 1)
              == rowv[:, None]).astype(jnp.float32)
    gate = lax.dot_general(onehot, seq_ref[...], (((1,), (0,)), ((), ())),
                           preferred_element_type=jnp.float32)
    h = h * gate

    logits = lax.dot_general(h, lw_ref[...], dn,
                             preferred_element_type=jnp.float32) + lb_ref[...]
    m = jnp.max(logits, axis=1, keepdims=True)
    e = jnp.exp(logits - m)
    out_ref[...] = (logits - m) - jnp.log(jnp.sum(e, axis=1, keepdims=True))


def kernel(x, edge_index, seq_reverse, seqid_reverse, row, ln_g, ln_b,
           llin_W, llin_b, rlin_W, rlin_b, gru_Wih, gru_Whh, gru_bih,
           gru_bhh, sage_Wl, sage_bl, sage_Wr, lin_W, lin_b):
    xcat = jnp.concatenate([x[:, :FH], x[:, FH:]], axis=0)
    src = edge_index[0]
    dst = edge_index[1]
    zf = jnp.zeros((N, FH), jnp.float32)

    agg, deg = _make_sc_segsum()(xcat, src, dst, zf)

    seq_out = pl.pallas_call(
        _seq_body,
        out_shape=jax.ShapeDtypeStruct((L, F), jnp.float32),
        scratch_shapes=[
            pltpu.VMEM((L, 3 * H), jnp.float32),
            pltpu.VMEM((L, H), jnp.float32),
        ],
    )(seq_reverse, ln_g.reshape(1, F), ln_b.reshape(1, F),
      llin_W, llin_b.reshape(1, H), rlin_W, rlin_b.reshape(1, H),
      gru_Wih, gru_Whh, gru_bih.reshape(1, 3 * H), gru_bhh.reshape(1, 3 * H))

    row3d = row.reshape(NBN, 1, BN)
    full = lambda *shape: pl.BlockSpec(shape, lambda i: (0,) * len(shape))
    out = pl.pallas_call(
        _fin_body,
        grid=(NBN,),
        in_specs=[
            pl.BlockSpec((BN, F), lambda i: (i, 0)),
            pl.BlockSpec((2, BN, FH), lambda i: (0, i, 0)),
            pl.BlockSpec((2, BN, FH), lambda i: (0, i, 0)),
            pl.BlockSpec((1, 1, BN), lambda i: (i, 0, 0)),
            full(L, F),
            full(H, F),
            full(1, H),
            full(H, F),
            full(C, H),
            full(1, C),
        ],
        out_specs=pl.BlockSpec((BN, C), lambda i: (i, 0)),
        out_shape=jax.ShapeDtypeStruct((N, C), jnp.float32),
    )(x, agg, deg, row3d, seq_out, sage_Wl, sage_bl.reshape(1, H),
      sage_Wr, lin_W, lin_b.reshape(1, C))
    return out


# speedup vs baseline: 4.0515x; 1.2565x over previous
import functools

import jax
import jax.numpy as jnp
from jax import lax
from jax.experimental import pallas as pl
from jax.experimental.pallas import tpu as pltpu
from jax.experimental.pallas import tpu_sc as plsc

N, E, F, H, C, L = 10000, 160000, 256, 256, 40, 1024
FH = F // 2
NSUB = 16
EPS = E // NSUB
BLK = 128
NFB = EPS // BLK
TAIL = EPS - NFB * BLK
CHUNK = 2 * EPS
ROWS_A = 640
ROWS_TAIL = N - (NSUB - 1) * ROWS_A
DEG0 = NFB // 2


@functools.cache
def _make_sc_segsum():
    mesh = plsc.VectorSubcoreMesh(core_axis_name="c", subcore_axis_name="s")
    return functools.partial(
        pl.kernel,
        out_type=(
            jax.ShapeDtypeStruct((2, N, FH), jnp.float32),
            jax.ShapeDtypeStruct((2, N, FH), jnp.float32),
        ),
        mesh=mesh,
        scratch_types=[
            pltpu.VMEM((2 * BLK,), jnp.int32),
            pltpu.VMEM((BLK,), jnp.int32),
            pltpu.VMEM((TAIL,), jnp.int32),
            pltpu.VMEM((BLK, FH), jnp.float32),
            pltpu.VMEM((BLK, FH), jnp.float32),
            pltpu.VMEM_SHARED((N, FH), jnp.float32),
            pltpu.SemaphoreType.DMA,
        ],
    )(_sc_segsum_body)


def _sc_segsum_body(xcat_hbm, eint_hbm, zf_hbm,
                    agg_hbm, deg_hbm,
                    sidi_v, di_v, dit_v, rows_v, ones_v, acc_sh, sem):
    c = lax.axis_index("c")
    s = lax.axis_index("s")
    row0 = pl.multiple_of(s * ROWS_A, 8)

    def zero_my_slice():
        @pl.when(s < NSUB - 1)
        def _():
            pltpu.sync_copy(zf_hbm.at[pl.ds(row0, ROWS_A)],
                            acc_sh.at[pl.ds(row0, ROWS_A)])

        @pl.when(s == NSUB - 1)
        def _():
            pltpu.sync_copy(zf_hbm.at[pl.ds(N - ROWS_TAIL, ROWS_TAIL)],
                            acc_sh.at[pl.ds(N - ROWS_TAIL, ROWS_TAIL)])

    def copy_out_my_slice(out):
        @pl.when(s < NSUB - 1)
        def _():
            pltpu.sync_copy(acc_sh.at[pl.ds(row0, ROWS_A)],
                            out.at[c, pl.ds(row0, ROWS_A)])

        @pl.when(s == NSUB - 1)
        def _():
            pltpu.sync_copy(acc_sh.at[pl.ds(N - ROWS_TAIL, ROWS_TAIL)],
                            out.at[c, pl.ds(N - ROWS_TAIL, ROWS_TAIL)])

    zero_my_slice()

    def _init_ones(i, carry):
        def _in2(j, carry2):
            ones_v[i, pl.ds(j * 16, 16)] = jnp.ones((16,), jnp.float32)
            return carry2
        return lax.fori_loop(0, FH // 16, _in2, carry)
    lax.fori_loop(0, BLK, _init_ones, 0)

    plsc.subcore_barrier()
    e0 = s * CHUNK
    coff = c * N

    def _off_si(nvec, carry0):
        def _off(kk, carry2):
            sidi_v[pl.ds(kk * 16, 16)] = (sidi_v[pl.ds(kk * 16, 16)]
                                          + coff * jnp.ones((16,), jnp.int32))
            return carry2
        return lax.fori_loop(0, nvec, _off, carry0)

    def _copy_di(nvec, carry0):
        def _cp(kk, carry2):
            di_v[pl.ds(kk * 16, 16)] = sidi_v[pl.ds(BLK + kk * 16, 16)]
            return carry2
        return lax.fori_loop(0, nvec, _cp, carry0)

    def _blk(b, carry):
        pltpu.sync_copy(eint_hbm.at[pl.ds(e0 + b * (2 * BLK), 2 * BLK)], sidi_v)
        _off_si(BLK // 16, 0)
        _copy_di(BLK // 16, 0)
        pltpu.async_copy(xcat_hbm.at[sidi_v.at[pl.ds(0, BLK)]], rows_v,
                         sem).wait()
        pltpu.sync_copy(rows_v, acc_sh.at[di_v], add=True)
        return carry

    lax.fori_loop(0, NFB, _blk, 0)

    t0 = e0 + NFB * 2 * BLK
    pltpu.sync_copy(eint_hbm.at[pl.ds(t0, 2 * TAIL)], sidi_v.at[pl.ds(0, 32)])
    _off_si(TAIL // 16, 0)
    dit_v[pl.ds(0, 16)] = sidi_v[pl.ds(TAIL, 16)]
    pltpu.async_copy(xcat_hbm.at[sidi_v.at[pl.ds(0, TAIL)]],
                     rows_v.at[pl.ds(0, TAIL)], sem).wait()
    pltpu.sync_copy(rows_v.at[pl.ds(0, TAIL)],
                    acc_sh.at[dit_v], add=True)

    plsc.subcore_barrier()
    copy_out_my_slice(agg_hbm)
    zero_my_slice()
    plsc.subcore_barrier()

    def _dblk(b, carry):
        be = b + DEG0 * c
        pltpu.sync_copy(eint_hbm.at[pl.ds(e0 + be * (2 * BLK) + BLK, BLK)],
                        di_v)
        pltpu.sync_copy(ones_v, acc_sh.at[di_v], add=True)
        return carry

    lax.fori_loop(0, DEG0, _dblk, 0)

    @pl.when(c == 0)
    def _():
        pltpu.sync_copy(eint_hbm.at[pl.ds(t0 + TAIL, TAIL)], dit_v)
        pltpu.sync_copy(ones_v.at[pl.ds(0, TAIL)],
                        acc_sh.at[dit_v], add=True)

    plsc.subcore_barrier()
    copy_out_my_slice(deg_hbm)


def _seq_body(seq_ref, lng_ref, lnb_ref, lw_ref, lb_ref, rw_ref, rb_ref,
              wih_ref, whh_ref, bih_ref, bhh_ref, out_ref, gi_ref, l_ref):
    sv = seq_ref[...]
    mu = jnp.mean(sv, axis=-1, keepdims=True)
    var = jnp.mean((sv - mu) * (sv - mu), axis=-1, keepdims=True)
    sn = (sv - mu) / jnp.sqrt(var + 1e-5) * lng_ref[...] + lnb_ref[...]

    dn = (((1,), (1,)), ((), ()))
    lin = lax.dot_general(sn, lw_ref[...], dn,
                          preferred_element_type=jnp.float32) + lb_ref[...]
    l_ref[...] = lin * 0.5 * (1.0 + lax.erf(lin * 0.7071067811865475))
    rin = lax.dot_general(sn, rw_ref[...], dn,
                          preferred_element_type=jnp.float32) + rb_ref[...]
    gi_ref[...] = lax.dot_general(rin, wih_ref[...], dn,
                                  preferred_element_type=jnp.float32) + bih_ref[...]

    whh = whh_ref[...]
    bhh = bhh_ref[...]

    def _step(t, h):
        gi = gi_ref[pl.ds(t, 1), :]
        gh = lax.dot_general(h, whh, dn,
                             preferred_element_type=jnp.float32) + bhh
        r = jax.nn.sigmoid(gi[:, 0:H] + gh[:, 0:H])
        z = jax.nn.sigmoid(gi[:, H:2 * H] + gh[:, H:2 * H])
        n = jnp.tanh(gi[:, 2 * H:] + r * gh[:, 2 * H:])
        hn = (1.0 - z) * n + z * h
        out_ref[pl.ds(t, 1), :] = l_ref[pl.ds(t, 1), :] * hn
        return hn

    lax.fori_loop(0, L, _step, jnp.zeros((1, H), jnp.float32))


BN = 400
NBN = N // BN


def _fin_body(x_ref, agg_ref, deg_ref, row_ref, seq_ref, wl_ref, bl_ref,
              wr_ref, lw_ref, lb_ref, out_ref):
    deg = deg_ref[0, :, 0:1] + deg_ref[1, :, 0:1]
    recip = 1.0 / jnp.maximum(deg, 1.0)
    m0 = agg_ref[0] * recip
    m1 = agg_ref[1] * recip
    dn = (((1,), (1,)), ((), ()))
    wl = wl_ref[...]
    h = (lax.dot_general(m0, wl[:, :FH], dn, preferred_element_type=jnp.float32)
         + lax.dot_general(m1, wl[:, FH:], dn, preferred_element_type=jnp.float32)
         + bl_ref[...]
         + lax.dot_general(x_ref[...], wr_ref[...], dn,
                           preferred_element_type=jnp.float32))
    h = jnp.maximum(h, 0.0)

    rowv = row_ref[0, 0, :]
    onehot = (lax.broadcasted_iota(jnp.int32, (BN, L), 1)
              == rowv[:, None]).astype(jnp.float32)
    gate = lax.dot_general(onehot, seq_ref[...], (((1,), (0,)), ((), ())),
                           preferred_element_type=jnp.float32)
    h = h * gate

    logits = lax.dot_general(h, lw_ref[...], dn,
                             preferred_element_type=jnp.float32) + lb_ref[...]
    m = jnp.max(logits, axis=1, keepdims=True)
    e = jnp.exp(logits - m)
    out_ref[...] = (logits - m) - jnp.log(jnp.sum(e, axis=1, keepdims=True))


def kernel(x, edge_index, seq_reverse, seqid_reverse, row, ln_g, ln_b,
           llin_W, llin_b, rlin_W, rlin_b, gru_Wih, gru_Whh, gru_bih,
           gru_bhh, sage_Wl, sage_bl, sage_Wr, lin_W, lin_b):
    xcat = jnp.concatenate([x[:, :FH], x[:, FH:]], axis=0)
    src_c = edge_index[0].reshape(NSUB, EPS)
    dst_c = edge_index[1].reshape(NSUB, EPS)
    body = jnp.concatenate([
        src_c[:, :NFB * BLK].reshape(NSUB, NFB, BLK),
        dst_c[:, :NFB * BLK].reshape(NSUB, NFB, BLK),
    ], axis=2).reshape(NSUB, NFB * 2 * BLK)
    tail = jnp.concatenate([src_c[:, NFB * BLK:], dst_c[:, NFB * BLK:]], axis=1)
    eint = jnp.concatenate([body, tail], axis=1).reshape(-1)
    zf = jnp.zeros((N, FH), jnp.float32)

    agg, deg = _make_sc_segsum()(xcat, eint, zf)

    seq_out = pl.pallas_call(
        _seq_body,
        out_shape=jax.ShapeDtypeStruct((L, F), jnp.float32),
        scratch_shapes=[
            pltpu.VMEM((L, 3 * H), jnp.float32),
            pltpu.VMEM((L, H), jnp.float32),
        ],
    )(seq_reverse, ln_g.reshape(1, F), ln_b.reshape(1, F),
      llin_W, llin_b.reshape(1, H), rlin_W, rlin_b.reshape(1, H),
      gru_Wih, gru_Whh, gru_bih.reshape(1, 3 * H), gru_bhh.reshape(1, 3 * H))

    row3d = row.reshape(NBN, 1, BN)
    full = lambda *shape: pl.BlockSpec(shape, lambda i: (0,) * len(shape))
    out = pl.pallas_call(
        _fin_body,
        grid=(NBN,),
        in_specs=[
            pl.BlockSpec((BN, F), lambda i: (i, 0)),
            pl.BlockSpec((2, BN, FH), lambda i: (0, i, 0)),
            pl.BlockSpec((2, BN, FH), lambda i: (0, i, 0)),
            pl.BlockSpec((1, 1, BN), lambda i: (i, 0, 0)),
            full(L, F),
            full(H, F),
            full(1, H),
            full(H, F),
            full(C, H),
            full(1, C),
        ],
        out_specs=pl.BlockSpec((BN, C), lambda i: (i, 0)),
        out_shape=jax.ShapeDtypeStruct((N, C), jnp.float32),
    )(x, agg, deg, row3d, seq_out, sage_Wl, sage_bl.reshape(1, H),
      sage_Wr, lin_W, lin_b.reshape(1, C))
    return out


# speedup vs baseline: 4.0524x; 1.0002x over previous
import functools

import jax
import jax.numpy as jnp
from jax import lax
from jax.experimental import pallas as pl
from jax.experimental.pallas import tpu as pltpu
from jax.experimental.pallas import tpu_sc as plsc

N, E, F, H, C, L = 10000, 160000, 256, 256, 40, 1024
FH = F // 2
NSUB = 16
EPS = E // NSUB
BLK = 128
NFB = EPS // BLK
TAIL = EPS - NFB * BLK
CHUNK = 2 * EPS
ROWS_A = 640
ROWS_TAIL = N - (NSUB - 1) * ROWS_A
DEG0 = NFB // 2


@functools.cache
def _make_sc_segsum():
    mesh = plsc.VectorSubcoreMesh(core_axis_name="c", subcore_axis_name="s")
    return functools.partial(
        pl.kernel,
        out_type=(
            jax.ShapeDtypeStruct((2, N, FH), jnp.float32),
            jax.ShapeDtypeStruct((2, N, FH), jnp.float32),
        ),
        mesh=mesh,
        scratch_types=[
            pltpu.VMEM((2 * BLK,), jnp.int32),
            pltpu.VMEM((2 * BLK,), jnp.int32),
            pltpu.VMEM((BLK,), jnp.int32),
            pltpu.VMEM((BLK,), jnp.int32),
            pltpu.VMEM((TAIL,), jnp.int32),
            pltpu.VMEM((BLK, FH), jnp.float32),
            pltpu.VMEM((BLK, FH), jnp.float32),
            pltpu.VMEM_SHARED((N, FH), jnp.float32),
            pltpu.SemaphoreType.DMA,
            pltpu.SemaphoreType.DMA,
            pltpu.SemaphoreType.DMA,
            pltpu.SemaphoreType.DMA,
            pltpu.SemaphoreType.DMA,
        ],
    )(_sc_segsum_body)


def _sc_segsum_body(xcat_hbm, eint_hbm, zf_hbm,
                    agg_hbm, deg_hbm,
                    sidi_a, sidi_b, di_a, di_b, dit_v, rows_a, rows_b,
                    acc_sh, semG, semI_a, semI_b, semS_a, semS_b):
    c = lax.axis_index("c")
    s = lax.axis_index("s")
    row0 = pl.multiple_of(s * ROWS_A, 8)

    def zero_my_slice():
        @pl.when(s < NSUB - 1)
        def _():
            pltpu.sync_copy(zf_hbm.at[pl.ds(row0, ROWS_A)],
                            acc_sh.at[pl.ds(row0, ROWS_A)])

        @pl.when(s == NSUB - 1)
        def _():
            pltpu.sync_copy(zf_hbm.at[pl.ds(N - ROWS_TAIL, ROWS_TAIL)],
                            acc_sh.at[pl.ds(N - ROWS_TAIL, ROWS_TAIL)])

    def copy_out_my_slice(out):
        @pl.when(s < NSUB - 1)
        def _():
            pltpu.sync_copy(acc_sh.at[pl.ds(row0, ROWS_A)],
                            out.at[c, pl.ds(row0, ROWS_A)])

        @pl.when(s == NSUB - 1)
        def _():
            pltpu.sync_copy(acc_sh.at[pl.ds(N - ROWS_TAIL, ROWS_TAIL)],
                            out.at[c, pl.ds(N - ROWS_TAIL, ROWS_TAIL)])

    zero_my_slice()

    e0 = s * CHUNK
    coff = c * N

    def _off_si(sidi_v, nvec, carry0):
        def _off(kk, carry2):
            sidi_v[pl.ds(kk * 16, 16)] = (sidi_v[pl.ds(kk * 16, 16)]
                                          + coff * jnp.ones((16,), jnp.int32))
            return carry2
        return lax.fori_loop(0, nvec, _off, carry0)

    def _copy_di(sidi_v, di_v, nvec, carry0):
        def _cp(kk, carry2):
            di_v[pl.ds(kk * 16, 16)] = sidi_v[pl.ds(BLK + kk * 16, 16)]
            return carry2
        return lax.fori_loop(0, nvec, _cp, carry0)

    def _idx_load(b, sidi_v, semI):
        return pltpu.async_copy(
            eint_hbm.at[pl.ds(e0 + b * (2 * BLK), 2 * BLK)], sidi_v, semI)

    _idx_load(0, sidi_a, semI_a)
    _idx_load(1, sidi_b, semI_b)

    def _pair(k, carry):
        def _one(b, sidi_v, di_v, rows_v, semI, semS):
            pltpu.make_async_copy(eint_hbm.at[pl.ds(0, 2 * BLK)], sidi_v,
                                  semI).wait()

            @pl.when(k > 0)
            def _():
                pltpu.make_async_copy(rows_v, acc_sh.at[di_v], semS).wait()

            _off_si(sidi_v, BLK // 16, 0)
            _copy_di(sidi_v, di_v, BLK // 16, 0)
            pltpu.async_copy(xcat_hbm.at[sidi_v.at[pl.ds(0, BLK)]], rows_v,
                             semG).wait()

            @pl.when(k < NFB // 2 - 1)
            def _():
                _idx_load(b + 2, sidi_v, semI)

            pltpu.async_copy(rows_v, acc_sh.at[di_v], semS, add=True)

        _one(2 * k, sidi_a, di_a, rows_a, semI_a, semS_a)
        _one(2 * k + 1, sidi_b, di_b, rows_b, semI_b, semS_b)
        return carry

    lax.fori_loop(0, NFB // 2, _pair, 0)
    pltpu.make_async_copy(rows_a, acc_sh.at[di_a], semS_a).wait()
    pltpu.make_async_copy(rows_b, acc_sh.at[di_b], semS_b).wait()

    t0 = e0 + NFB * 2 * BLK
    pltpu.sync_copy(eint_hbm.at[pl.ds(t0, 2 * TAIL)], sidi_a.at[pl.ds(0, 32)])
    _off_si(sidi_a, TAIL // 16, 0)
    dit_v[pl.ds(0, 16)] = sidi_a[pl.ds(TAIL, 16)]
    pltpu.async_copy(xcat_hbm.at[sidi_a.at[pl.ds(0, TAIL)]],
                     rows_a.at[pl.ds(0, TAIL)], semG).wait()
    pltpu.sync_copy(rows_a.at[pl.ds(0, TAIL)],
                    acc_sh.at[dit_v], add=True)

    plsc.subcore_barrier()
    copy_out_my_slice(agg_hbm)
    zero_my_slice()

    def _init_ones(i, carry):
        def _in2(j, carry2):
            rows_a[i, pl.ds(j * 16, 16)] = jnp.ones((16,), jnp.float32)
            return carry2
        return lax.fori_loop(0, FH // 16, _in2, carry)
    lax.fori_loop(0, BLK, _init_ones, 0)
    plsc.subcore_barrier()

    def _done(k, b, di_v, semS):
        @pl.when(k > 0)
        def _():
            pltpu.make_async_copy(rows_a, acc_sh.at[di_v], semS).wait()

        be = b + DEG0 * c
        pltpu.sync_copy(eint_hbm.at[pl.ds(e0 + be * (2 * BLK) + BLK, BLK)],
                        di_v)
        pltpu.async_copy(rows_a, acc_sh.at[di_v], semS, add=True)

    def _dpair(k, carry):
        _done(k, 2 * k, di_a, semS_a)

        @pl.when(2 * k + 1 < DEG0)
        def _():
            _done(k, 2 * k + 1, di_b, semS_b)
        return carry

    lax.fori_loop(0, (DEG0 + 1) // 2, _dpair, 0)
    pltpu.make_async_copy(rows_a, acc_sh.at[di_a], semS_a).wait()
    pltpu.make_async_copy(rows_a, acc_sh.at[di_b], semS_b).wait()

    @pl.when(c == 0)
    def _():
        pltpu.sync_copy(eint_hbm.at[pl.ds(t0 + TAIL, TAIL)], dit_v)
        pltpu.sync_copy(rows_a.at[pl.ds(0, TAIL)],
                        acc_sh.at[dit_v], add=True)

    plsc.subcore_barrier()
    copy_out_my_slice(deg_hbm)


def _seq_body(seq_ref, lng_ref, lnb_ref, lw_ref, lb_ref, rw_ref, rb_ref,
              wih_ref, whh_ref, bih_ref, bhh_ref, out_ref, gi_ref, l_ref):
    sv = seq_ref[...]
    mu = jnp.mean(sv, axis=-1, keepdims=True)
    var = jnp.mean((sv - mu) * (sv - mu), axis=-1, keepdims=True)
    sn = (sv - mu) / jnp.sqrt(var + 1e-5) * lng_ref[...] + lnb_ref[...]

    dn = (((1,), (1,)), ((), ()))
    lin = lax.dot_general(sn, lw_ref[...], dn,
                          preferred_element_type=jnp.float32) + lb_ref[...]
    l_ref[...] = lin * 0.5 * (1.0 + lax.erf(lin * 0.7071067811865475))
    rin = lax.dot_general(sn, rw_ref[...], dn,
                          preferred_element_type=jnp.float32) + rb_ref[...]
    gi_ref[...] = lax.dot_general(rin, wih_ref[...], dn,
                                  preferred_element_type=jnp.float32) + bih_ref[...]

    whh = whh_ref[...]
    bhh = bhh_ref[...]

    def _step(t, h):
        gi = gi_ref[pl.ds(t, 1), :]
        gh = lax.dot_general(h, whh, dn,
                             preferred_element_type=jnp.float32) + bhh
        r = jax.nn.sigmoid(gi[:, 0:H] + gh[:, 0:H])
        z = jax.nn.sigmoid(gi[:, H:2 * H] + gh[:, H:2 * H])
        n = jnp.tanh(gi[:, 2 * H:] + r * gh[:, 2 * H:])
        hn = (1.0 - z) * n + z * h
        out_ref[pl.ds(t, 1), :] = l_ref[pl.ds(t, 1), :] * hn
        return hn

    lax.fori_loop(0, L, _step, jnp.zeros((1, H), jnp.float32))


BN = 400
NBN = N // BN


def _fin_body(x_ref, agg_ref, deg_ref, row_ref, seq_ref, wl_ref, bl_ref,
              wr_ref, lw_ref, lb_ref, out_ref):
    deg = deg_ref[0, :, 0:1] + deg_ref[1, :, 0:1]
    recip = 1.0 / jnp.maximum(deg, 1.0)
    m0 = agg_ref[0] * recip
    m1 = agg_ref[1] * recip
    dn = (((1,), (1,)), ((), ()))
    wl = wl_ref[...]
    h = (lax.dot_general(m0, wl[:, :FH], dn, preferred_element_type=jnp.float32)
         + lax.dot_general(m1, wl[:, FH:], dn, preferred_element_type=jnp.float32)
         + bl_ref[...]
         + lax.dot_general(x_ref[...], wr_ref[...], dn,
                           preferred_element_type=jnp.float32))
    h = jnp.maximum(h, 0.0)

    rowv = row_ref[0, 0, :]
    onehot = (lax.broadcasted_iota(jnp.int32, (BN, L), 1)
              == rowv[:, None]).astype(jnp.float32)
    gate = lax.dot_general(onehot, seq_ref[...], (((1,), (0,)), ((), ())),
                           preferred_element_type=jnp.float32)
    h = h * gate

    logits = lax.dot_general(h, lw_ref[...], dn,
                             preferred_element_type=jnp.float32) + lb_ref[...]
    m = jnp.max(logits, axis=1, keepdims=True)
    e = jnp.exp(logits - m)
    out_ref[...] = (logits - m) - jnp.log(jnp.sum(e, axis=1, keepdims=True))


def kernel(x, edge_index, seq_reverse, seqid_reverse, row, ln_g, ln_b,
           llin_W, llin_b, rlin_W, rlin_b, gru_Wih, gru_Whh, gru_bih,
           gru_bhh, sage_Wl, sage_bl, sage_Wr, lin_W, lin_b):
    xcat = jnp.concatenate([x[:, :FH], x[:, FH:]], axis=0)
    src_c = edge_index[0].reshape(NSUB, EPS)
    dst_c = edge_index[1].reshape(NSUB, EPS)
    body = jnp.concatenate([
        src_c[:, :NFB * BLK].reshape(NSUB, NFB, BLK),
        dst_c[:, :NFB * BLK].reshape(NSUB, NFB, BLK),
    ], axis=2).reshape(NSUB, NFB * 2 * BLK)
    tail = jnp.concatenate([src_c[:, NFB * BLK:], dst_c[:, NFB * BLK:]], axis=1)
    eint = jnp.concatenate([body, tail], axis=1).reshape(-1)
    zf = jnp.zeros((N, FH), jnp.float32)

    agg, deg = _make_sc_segsum()(xcat, eint, zf)

    seq_out = pl.pallas_call(
        _seq_body,
        out_shape=jax.ShapeDtypeStruct((L, F), jnp.float32),
        scratch_shapes=[
            pltpu.VMEM((L, 3 * H), jnp.float32),
            pltpu.VMEM((L, H), jnp.float32),
        ],
    )(seq_reverse, ln_g.reshape(1, F), ln_b.reshape(1, F),
      llin_W, llin_b.reshape(1, H), rlin_W, rlin_b.reshape(1, H),
      gru_Wih, gru_Whh, gru_bih.reshape(1, 3 * H), gru_bhh.reshape(1, 3 * H))

    row3d = row.reshape(NBN, 1, BN)
    full = lambda *shape: pl.BlockSpec(shape, lambda i: (0,) * len(shape))
    out = pl.pallas_call(
        _fin_body,
        grid=(NBN,),
        in_specs=[
            pl.BlockSpec((BN, F), lambda i: (i, 0)),
            pl.BlockSpec((2, BN, FH), lambda i: (0, i, 0)),
            pl.BlockSpec((2, BN, FH), lambda i: (0, i, 0)),
            pl.BlockSpec((1, 1, BN), lambda i: (i, 0, 0)),
            full(L, F),
            full(H, F),
            full(1, H),
            full(H, F),
            full(C, H),
            full(1, C),
        ],
        out_specs=pl.BlockSpec((BN, C), lambda i: (i, 0)),
        out_shape=jax.ShapeDtypeStruct((N, C), jnp.float32),
    )(x, agg, deg, row3d, seq_out, sage_Wl, sage_bl.reshape(1, H),
      sage_Wr, lin_W, lin_b.reshape(1, C))
    return out


# speedup vs baseline: 4.1157x; 1.0156x over previous
import functools

import jax
import jax.numpy as jnp
from jax import lax
from jax.experimental import pallas as pl
from jax.experimental.pallas import tpu as pltpu
from jax.experimental.pallas import tpu_sc as plsc

N, E, F, H, C, L = 10000, 160000, 256, 256, 40, 1024
FH = F // 2
NSUB = 16
EPS = E // NSUB
BLK = 128
NFB = EPS // BLK
TAIL = EPS - NFB * BLK
CHUNK = 2 * EPS
ROWS_A = 640
ROWS_TAIL = N - (NSUB - 1) * ROWS_A
DEG0 = NFB // 2


@functools.cache
def _make_sc_segsum():
    mesh = plsc.VectorSubcoreMesh(core_axis_name="c", subcore_axis_name="s")
    return functools.partial(
        pl.kernel,
        out_type=(
            jax.ShapeDtypeStruct((2, N, FH), jnp.float32),
            jax.ShapeDtypeStruct((2, N, FH), jnp.float32),
        ),
        mesh=mesh,
        scratch_types=[
            pltpu.VMEM((2 * BLK,), jnp.int32),
            pltpu.VMEM((2 * BLK,), jnp.int32),
            pltpu.VMEM((BLK,), jnp.int32),
            pltpu.VMEM((BLK,), jnp.int32),
            pltpu.VMEM((TAIL,), jnp.int32),
            pltpu.VMEM((BLK, FH), jnp.float32),
            pltpu.VMEM((BLK, FH), jnp.float32),
            pltpu.VMEM_SHARED((N, FH), jnp.float32),
            pltpu.SemaphoreType.DMA,
            pltpu.SemaphoreType.DMA,
            pltpu.SemaphoreType.DMA,
            pltpu.SemaphoreType.DMA,
            pltpu.SemaphoreType.DMA,
        ],
    )(_sc_segsum_body)


def _fill(ref, nrows, val):
    def _fi(i, carry):
        def _fj(j, carry2):
            ref[i, pl.ds(j * 16, 16)] = jnp.full((16,), val, jnp.float32)
            return carry2
        return lax.fori_loop(0, FH // 16, _fj, carry)
    lax.fori_loop(0, nrows, _fi, 0)


def _sc_segsum_body(x2_hbm, eint_hbm,
                    agg_hbm, deg_hbm,
                    sidi_a, sidi_b, di_a, di_b, dit_v, rows_a, rows_b,
                    acc_sh, semG, semI_a, semI_b, semS_a, semS_b):
    c = lax.axis_index("c")
    s = lax.axis_index("s")
    row0 = pl.multiple_of(s * ROWS_A, 8)

    def zero_my_slice(zbuf):
        @pl.when(s < NSUB - 1)
        def _():
            def _z(j, carry):
                zj = pl.multiple_of(row0 + j * BLK, 8)
                pltpu.sync_copy(zbuf, acc_sh.at[pl.ds(zj, BLK)])
                return carry
            lax.fori_loop(0, ROWS_A // BLK, _z, 0)

        @pl.when(s == NSUB - 1)
        def _():
            def _z(j, carry):
                zj = pl.multiple_of(N - ROWS_TAIL + j * BLK, 8)
                pltpu.sync_copy(zbuf, acc_sh.at[pl.ds(zj, BLK)])
                return carry
            lax.fori_loop(0, ROWS_TAIL // BLK, _z, 0)
            pltpu.sync_copy(zbuf.at[pl.ds(0, ROWS_TAIL % BLK)],
                            acc_sh.at[pl.ds(N - (ROWS_TAIL % BLK),
                                            ROWS_TAIL % BLK)])

    def copy_out_my_slice(out, cols):
        del cols
        @pl.when(s < NSUB - 1)
        def _():
            pltpu.sync_copy(acc_sh.at[pl.ds(row0, ROWS_A)],
                            out.at[c, pl.ds(row0, ROWS_A)])

        @pl.when(s == NSUB - 1)
        def _():
            pltpu.sync_copy(acc_sh.at[pl.ds(N - ROWS_TAIL, ROWS_TAIL)],
                            out.at[c, pl.ds(N - ROWS_TAIL, ROWS_TAIL)])

    _fill(rows_a, BLK, 0.0)
    zero_my_slice(rows_a)

    e0 = s * CHUNK

    def _off_si(sidi_v, nvec, carry0):
        def _off(kk, carry2):
            sidi_v[pl.ds(kk * 16, 16)] = (
                sidi_v[pl.ds(kk * 16, 16)] * 2
                + c * jnp.ones((16,), jnp.int32))
            return carry2
        return lax.fori_loop(0, nvec, _off, carry0)

    def _copy_di(sidi_v, di_v, nvec, carry0):
        def _cp(kk, carry2):
            di_v[pl.ds(kk * 16, 16)] = sidi_v[pl.ds(BLK + kk * 16, 16)]
            return carry2
        return lax.fori_loop(0, nvec, _cp, carry0)

    def _idx_load(b, sidi_v, semI):
        return pltpu.async_copy(
            eint_hbm.at[pl.ds(e0 + b * (2 * BLK), 2 * BLK)], sidi_v, semI)

    _idx_load(0, sidi_a, semI_a)
    _idx_load(1, sidi_b, semI_b)

    def _pair(k, carry):
        def _one(b, sidi_v, di_v, rows_v, semI, semS):
            pltpu.make_async_copy(eint_hbm.at[pl.ds(0, 2 * BLK)], sidi_v,
                                  semI).wait()

            @pl.when(k > 0)
            def _():
                pltpu.make_async_copy(rows_v, acc_sh.at[di_v], semS).wait()

            _off_si(sidi_v, BLK // 16, 0)
            _copy_di(sidi_v, di_v, BLK // 16, 0)
            pltpu.async_copy(x2_hbm.at[sidi_v.at[pl.ds(0, BLK)]], rows_v,
                             semG).wait()

            @pl.when(k < NFB // 2 - 1)
            def _():
                _idx_load(b + 2, sidi_v, semI)

            pltpu.async_copy(rows_v, acc_sh.at[di_v], semS, add=True)

        _one(2 * k, sidi_a, di_a, rows_a, semI_a, semS_a)
        _one(2 * k + 1, sidi_b, di_b, rows_b, semI_b, semS_b)
        return carry

    lax.fori_loop(0, NFB // 2, _pair, 0)
    pltpu.make_async_copy(rows_a, acc_sh.at[di_a], semS_a).wait()
    pltpu.make_async_copy(rows_b, acc_sh.at[di_b], semS_b).wait()

    t0 = e0 + NFB * 2 * BLK
    pltpu.sync_copy(eint_hbm.at[pl.ds(t0, 2 * TAIL)], sidi_a.at[pl.ds(0, 32)])
    _off_si(sidi_a, TAIL // 16, 0)
    dit_v[pl.ds(0, 16)] = sidi_a[pl.ds(TAIL, 16)]
    pltpu.async_copy(x2_hbm.at[sidi_a.at[pl.ds(0, TAIL)]],
                     rows_a.at[pl.ds(0, TAIL)], semG).wait()
    pltpu.sync_copy(rows_a.at[pl.ds(0, TAIL)],
                    acc_sh.at[dit_v], add=True)

    plsc.subcore_barrier()
    copy_out_my_slice(agg_hbm, FH)
    _fill(rows_b, BLK, 0.0)
    zero_my_slice(rows_b)
    _fill(rows_a, BLK, 1.0)
    plsc.subcore_barrier()

    def _done(k, b, di_v, semS):
        @pl.when(k > 0)
        def _():
            pltpu.make_async_copy(rows_a, acc_sh.at[di_v], semS).wait()

        be = b + DEG0 * c
        pltpu.sync_copy(eint_hbm.at[pl.ds(e0 + be * (2 * BLK) + BLK, BLK)],
                        di_v)
        pltpu.async_copy(rows_a, acc_sh.at[di_v], semS, add=True)

    def _dpair(k, carry):
        _done(k, 2 * k, di_a, semS_a)

        @pl.when(2 * k + 1 < DEG0)
        def _():
            _done(k, 2 * k + 1, di_b, semS_b)
        return carry

    lax.fori_loop(0, (DEG0 + 1) // 2, _dpair, 0)
    pltpu.make_async_copy(rows_a, acc_sh.at[di_a], semS_a).wait()
    pltpu.make_async_copy(rows_a, acc_sh.at[di_b], semS_b).wait()

    @pl.when(c == 0)
    def _():
        pltpu.sync_copy(eint_hbm.at[pl.ds(t0 + TAIL, TAIL)], dit_v)
        pltpu.sync_copy(rows_a.at[pl.ds(0, TAIL)],
                        acc_sh.at[dit_v], add=True)

    plsc.subcore_barrier()
    copy_out_my_slice(deg_hbm, 16)


def _seq_body(seq_ref, lng_ref, lnb_ref, lw_ref, lb_ref, rw_ref, rb_ref,
              wih_ref, whh_ref, bih_ref, bhh_ref, out_ref, gi_ref, l_ref):
    sv = seq_ref[...]
    mu = jnp.mean(sv, axis=-1, keepdims=True)
    var = jnp.mean((sv - mu) * (sv - mu), axis=-1, keepdims=True)
    sn = (sv - mu) / jnp.sqrt(var + 1e-5) * lng_ref[...] + lnb_ref[...]

    dn = (((1,), (1,)), ((), ()))
    lin = lax.dot_general(sn, lw_ref[...], dn,
                          preferred_element_type=jnp.float32) + lb_ref[...]
    l_ref[...] = lin * 0.5 * (1.0 + lax.erf(lin * 0.7071067811865475))
    rin = lax.dot_general(sn, rw_ref[...], dn,
                          preferred_element_type=jnp.float32) + rb_ref[...]
    gi_ref[...] = lax.dot_general(rin, wih_ref[...], dn,
                                  preferred_element_type=jnp.float32) + bih_ref[...]

    whh = whh_ref[...]
    bhh = bhh_ref[...]

    def _step(t, h):
        gi = gi_ref[pl.ds(t, 1), :]
        gh = lax.dot_general(h, whh, dn,
                             preferred_element_type=jnp.float32) + bhh
        r = jax.nn.sigmoid(gi[:, 0:H] + gh[:, 0:H])
        z = jax.nn.sigmoid(gi[:, H:2 * H] + gh[:, H:2 * H])
        n = jnp.tanh(gi[:, 2 * H:] + r * gh[:, 2 * H:])
        hn = (1.0 - z) * n + z * h
        out_ref[pl.ds(t, 1), :] = l_ref[pl.ds(t, 1), :] * hn
        return hn

    lax.fori_loop(0, L, _step, jnp.zeros((1, H), jnp.float32))


BN = 400
NBN = N // BN


def _fin_body(x_ref, agg_ref, deg_ref, row_ref, seq_ref, wl_ref, bl_ref,
              wr_ref, lw_ref, lb_ref, out_ref):
    deg = deg_ref[0, :, 0:1] + deg_ref[1, :, 0:1]
    recip = 1.0 / jnp.maximum(deg, 1.0)
    m0 = agg_ref[0] * recip
    m1 = agg_ref[1] * recip
    dn = (((1,), (1,)), ((), ()))
    wl = wl_ref[...]
    h = (lax.dot_general(m0, wl[:, :FH], dn, preferred_element_type=jnp.float32)
         + lax.dot_general(m1, wl[:, FH:], dn, preferred_element_type=jnp.float32)
         + bl_ref[...]
         + lax.dot_general(x_ref[...], wr_ref[...], dn,
                           preferred_element_type=jnp.float32))
    h = jnp.maximum(h, 0.0)

    rowv = row_ref[0, 0, :]
    onehot = (lax.broadcasted_iota(jnp.int32, (BN, L), 1)
              == rowv[:, None]).astype(jnp.float32)
    gate = lax.dot_general(onehot, seq_ref[...], (((1,), (0,)), ((), ())),
                           preferred_element_type=jnp.float32)
    h = h * gate

    logits = lax.dot_general(h, lw_ref[...], dn,
                             preferred_element_type=jnp.float32) + lb_ref[...]
    m = jnp.max(logits, axis=1, keepdims=True)
    e = jnp.exp(logits - m)
    out_ref[...] = (logits - m) - jnp.log(jnp.sum(e, axis=1, keepdims=True))


def kernel(x, edge_index, seq_reverse, seqid_reverse, row, ln_g, ln_b,
           llin_W, llin_b, rlin_W, rlin_b, gru_Wih, gru_Whh, gru_bih,
           gru_bhh, sage_Wl, sage_bl, sage_Wr, lin_W, lin_b):
    x2 = x.reshape(2 * N, FH)
    src_c = edge_index[0].reshape(NSUB, EPS)
    dst_c = edge_index[1].reshape(NSUB, EPS)
    body = jnp.concatenate([
        src_c[:, :NFB * BLK].reshape(NSUB, NFB, BLK),
        dst_c[:, :NFB * BLK].reshape(NSUB, NFB, BLK),
    ], axis=2).reshape(NSUB, NFB * 2 * BLK)
    tail = jnp.concatenate([src_c[:, NFB * BLK:], dst_c[:, NFB * BLK:]], axis=1)
    eint = jnp.concatenate([body, tail], axis=1).reshape(-1)

    agg, deg = _make_sc_segsum()(x2, eint)

    seq_out = pl.pallas_call(
        _seq_body,
        out_shape=jax.ShapeDtypeStruct((L, F), jnp.float32),
        scratch_shapes=[
            pltpu.VMEM((L, 3 * H), jnp.float32),
            pltpu.VMEM((L, H), jnp.float32),
        ],
    )(seq_reverse, ln_g.reshape(1, F), ln_b.reshape(1, F),
      llin_W, llin_b.reshape(1, H), rlin_W, rlin_b.reshape(1, H),
      gru_Wih, gru_Whh, gru_bih.reshape(1, 3 * H), gru_bhh.reshape(1, 3 * H))

    row3d = row.reshape(NBN, 1, BN)
    full = lambda *shape: pl.BlockSpec(shape, lambda i: (0,) * len(shape))
    out = pl.pallas_call(
        _fin_body,
        grid=(NBN,),
        in_specs=[
            pl.BlockSpec((BN, F), lambda i: (i, 0)),
            pl.BlockSpec((2, BN, FH), lambda i: (0, i, 0)),
            pl.BlockSpec((2, BN, FH), lambda i: (0, i, 0)),
            pl.BlockSpec((1, 1, BN), lambda i: (i, 0, 0)),
            full(L, F),
            full(H, F),
            full(1, H),
            full(H, F),
            full(C, H),
            full(1, C),
        ],
        out_specs=pl.BlockSpec((BN, C), lambda i: (i, 0)),
        out_shape=jax.ShapeDtypeStruct((N, C), jnp.float32),
    )(x, agg, deg, row3d, seq_out, sage_Wl, sage_bl.reshape(1, H),
      sage_Wr, lin_W, lin_b.reshape(1, C))
    return out


# speedup vs baseline: 4.1160x; 1.0001x over previous
import functools

import jax
import jax.numpy as jnp
from jax import lax
from jax.experimental import pallas as pl
from jax.experimental.pallas import tpu as pltpu
from jax.experimental.pallas import tpu_sc as plsc

N, E, F, H, C, L = 10000, 160000, 256, 256, 40, 1024
FH = F // 2
NSUB = 16
EPS = E // NSUB
BLK = 128
NFB = EPS // BLK
TAIL = EPS - NFB * BLK
CHUNK = 2 * EPS
ROWS_A = 640
ROWS_TAIL = N - (NSUB - 1) * ROWS_A
DEG0 = NFB // 2


@functools.cache
def _make_sc_segsum():
    mesh = plsc.VectorSubcoreMesh(core_axis_name="c", subcore_axis_name="s")
    return functools.partial(
        pl.kernel,
        out_type=(
            jax.ShapeDtypeStruct((2, N, FH), jnp.float32),
            jax.ShapeDtypeStruct((2, N, FH), jnp.float32),
        ),
        mesh=mesh,
        scratch_types=[
            pltpu.VMEM((2 * BLK,), jnp.int32),
            pltpu.VMEM((2 * BLK,), jnp.int32),
            pltpu.VMEM((BLK,), jnp.int32),
            pltpu.VMEM((BLK,), jnp.int32),
            pltpu.VMEM((TAIL,), jnp.int32),
            pltpu.VMEM((BLK, FH), jnp.float32),
            pltpu.VMEM((BLK, FH), jnp.float32),
            pltpu.VMEM_SHARED((N, FH), jnp.float32),
            pltpu.SemaphoreType.DMA,
            pltpu.SemaphoreType.DMA,
            pltpu.SemaphoreType.DMA,
            pltpu.SemaphoreType.DMA,
            pltpu.SemaphoreType.DMA,
            pltpu.SemaphoreType.DMA,
        ],
    )(_sc_segsum_body)


def _fill(ref, nrows, val):
    def _fi(i, carry):
        def _fj(j, carry2):
            ref[i, pl.ds(j * 16, 16)] = jnp.full((16,), val, jnp.float32)
            return carry2
        return lax.fori_loop(0, FH // 16, _fj, carry)
    lax.fori_loop(0, nrows, _fi, 0)


def _sc_segsum_body(x2_hbm, eint_hbm,
                    agg_hbm, deg_hbm,
                    sidi_a, sidi_b, di_a, di_b, dit_v, rows_a, rows_b,
                    acc_sh, semG, semG_b, semI_a, semI_b, semS_a, semS_b):
    c = lax.axis_index("c")
    s = lax.axis_index("s")
    row0 = pl.multiple_of(s * ROWS_A, 8)

    def zero_my_slice(zbuf):
        @pl.when(s < NSUB - 1)
        def _():
            def _z(j, carry):
                zj = pl.multiple_of(row0 + j * BLK, 8)
                pltpu.sync_copy(zbuf, acc_sh.at[pl.ds(zj, BLK)])
                return carry
            lax.fori_loop(0, ROWS_A // BLK, _z, 0)

        @pl.when(s == NSUB - 1)
        def _():
            def _z(j, carry):
                zj = pl.multiple_of(N - ROWS_TAIL + j * BLK, 8)
                pltpu.sync_copy(zbuf, acc_sh.at[pl.ds(zj, BLK)])
                return carry
            lax.fori_loop(0, ROWS_TAIL // BLK, _z, 0)
            pltpu.sync_copy(zbuf.at[pl.ds(0, ROWS_TAIL % BLK)],
                            acc_sh.at[pl.ds(N - (ROWS_TAIL % BLK),
                                            ROWS_TAIL % BLK)])

    def copy_out_my_slice(out, cols):
        del cols
        @pl.when(s < NSUB - 1)
        def _():
            pltpu.sync_copy(acc_sh.at[pl.ds(row0, ROWS_A)],
                            out.at[c, pl.ds(row0, ROWS_A)])

        @pl.when(s == NSUB - 1)
        def _():
            pltpu.sync_copy(acc_sh.at[pl.ds(N - ROWS_TAIL, ROWS_TAIL)],
                            out.at[c, pl.ds(N - ROWS_TAIL, ROWS_TAIL)])

    _fill(rows_a, BLK, 0.0)
    zero_my_slice(rows_a)

    e0 = s * CHUNK

    def _off_si(sidi_v, nvec, carry0):
        def _off(kk, carry2):
            sidi_v[pl.ds(kk * 16, 16)] = (
                sidi_v[pl.ds(kk * 16, 16)] * 2
                + c * jnp.ones((16,), jnp.int32))
            return carry2
        return lax.fori_loop(0, nvec, _off, carry0)

    def _copy_di(sidi_v, di_v, nvec, carry0):
        def _cp(kk, carry2):
            di_v[pl.ds(kk * 16, 16)] = sidi_v[pl.ds(BLK + kk * 16, 16)]
            return carry2
        return lax.fori_loop(0, nvec, _cp, carry0)

    def _idx_load(b, sidi_v, semI):
        return pltpu.async_copy(
            eint_hbm.at[pl.ds(e0 + b * (2 * BLK), 2 * BLK)], sidi_v, semI)

    _idx_load(0, sidi_a, semI_a)
    _idx_load(1, sidi_b, semI_b)

    def _pair(k, carry):
        def _front(sidi_v, di_v, rows_v, semI, semS, semGx):
            pltpu.make_async_copy(eint_hbm.at[pl.ds(0, 2 * BLK)], sidi_v,
                                  semI).wait()

            @pl.when(k > 0)
            def _():
                pltpu.make_async_copy(rows_v, acc_sh.at[di_v], semS).wait()

            _off_si(sidi_v, BLK // 16, 0)
            _copy_di(sidi_v, di_v, BLK // 16, 0)
            pltpu.async_copy(x2_hbm.at[sidi_v.at[pl.ds(0, BLK)]], rows_v,
                             semGx)

        def _back(b, sidi_v, di_v, rows_v, semI, semS, semGx):
            pltpu.make_async_copy(x2_hbm.at[sidi_v.at[pl.ds(0, BLK)]], rows_v,
                                  semGx).wait()

            @pl.when(k < NFB // 2 - 1)
            def _():
                _idx_load(b + 2, sidi_v, semI)

            pltpu.async_copy(rows_v, acc_sh.at[di_v], semS, add=True)

        _front(sidi_a, di_a, rows_a, semI_a, semS_a, semG)
        _front(sidi_b, di_b, rows_b, semI_b, semS_b, semG_b)
        _back(2 * k, sidi_a, di_a, rows_a, semI_a, semS_a, semG)
        _back(2 * k + 1, sidi_b, di_b, rows_b, semI_b, semS_b, semG_b)
        return carry

    lax.fori_loop(0, NFB // 2, _pair, 0)
    pltpu.make_async_copy(rows_a, acc_sh.at[di_a], semS_a).wait()
    pltpu.make_async_copy(rows_b, acc_sh.at[di_b], semS_b).wait()

    t0 = e0 + NFB * 2 * BLK
    pltpu.sync_copy(eint_hbm.at[pl.ds(t0, 2 * TAIL)], sidi_a.at[pl.ds(0, 32)])
    _off_si(sidi_a, TAIL // 16, 0)
    dit_v[pl.ds(0, 16)] = sidi_a[pl.ds(TAIL, 16)]
    pltpu.async_copy(x2_hbm.at[sidi_a.at[pl.ds(0, TAIL)]],
                     rows_a.at[pl.ds(0, TAIL)], semG).wait()
    pltpu.sync_copy(rows_a.at[pl.ds(0, TAIL)],
                    acc_sh.at[dit_v], add=True)

    plsc.subcore_barrier()
    copy_out_my_slice(agg_hbm, FH)
    _fill(rows_b, BLK, 0.0)
    zero_my_slice(rows_b)
    _fill(rows_a, BLK, 1.0)
    plsc.subcore_barrier()

    def _done(k, b, di_v, semS):
        @pl.when(k > 0)
        def _():
            pltpu.make_async_copy(rows_a, acc_sh.at[di_v], semS).wait()

        be = b + DEG0 * c
        pltpu.sync_copy(eint_hbm.at[pl.ds(e0 + be * (2 * BLK) + BLK, BLK)],
                        di_v)
        pltpu.async_copy(rows_a, acc_sh.at[di_v], semS, add=True)

    def _dpair(k, carry):
        _done(k, 2 * k, di_a, semS_a)

        @pl.when(2 * k + 1 < DEG0)
        def _():
            _done(k, 2 * k + 1, di_b, semS_b)
        return carry

    lax.fori_loop(0, (DEG0 + 1) // 2, _dpair, 0)
    pltpu.make_async_copy(rows_a, acc_sh.at[di_a], semS_a).wait()
    pltpu.make_async_copy(rows_a, acc_sh.at[di_b], semS_b).wait()

    @pl.when(c == 0)
    def _():
        pltpu.sync_copy(eint_hbm.at[pl.ds(t0 + TAIL, TAIL)], dit_v)
        pltpu.sync_copy(rows_a.at[pl.ds(0, TAIL)],
                        acc_sh.at[dit_v], add=True)

    plsc.subcore_barrier()
    copy_out_my_slice(deg_hbm, 16)


def _seq_body(seq_ref, lng_ref, lnb_ref, lw_ref, lb_ref, rw_ref, rb_ref,
              wih_ref, whh_ref, bih_ref, bhh_ref, out_ref, gi_ref, l_ref):
    sv = seq_ref[...]
    mu = jnp.mean(sv, axis=-1, keepdims=True)
    var = jnp.mean((sv - mu) * (sv - mu), axis=-1, keepdims=True)
    sn = (sv - mu) / jnp.sqrt(var + 1e-5) * lng_ref[...] + lnb_ref[...]

    dn = (((1,), (1,)), ((), ()))
    lin = lax.dot_general(sn, lw_ref[...], dn,
                          preferred_element_type=jnp.float32) + lb_ref[...]
    l_ref[...] = lin * 0.5 * (1.0 + lax.erf(lin * 0.7071067811865475))
    rin = lax.dot_general(sn, rw_ref[...], dn,
                          preferred_element_type=jnp.float32) + rb_ref[...]
    gi_ref[...] = lax.dot_general(rin, wih_ref[...], dn,
                                  preferred_element_type=jnp.float32) + bih_ref[...]

    whh = whh_ref[...]
    bhh = bhh_ref[...]

    def _step(t, h):
        gi = gi_ref[pl.ds(t, 1), :]
        gh = lax.dot_general(h, whh, dn,
                             preferred_element_type=jnp.float32) + bhh
        r = jax.nn.sigmoid(gi[:, 0:H] + gh[:, 0:H])
        z = jax.nn.sigmoid(gi[:, H:2 * H] + gh[:, H:2 * H])
        n = jnp.tanh(gi[:, 2 * H:] + r * gh[:, 2 * H:])
        hn = (1.0 - z) * n + z * h
        out_ref[pl.ds(t, 1), :] = l_ref[pl.ds(t, 1), :] * hn
        return hn

    lax.fori_loop(0, L, _step, jnp.zeros((1, H), jnp.float32))


BN = 400
NBN = N // BN


def _fin_body(x_ref, agg_ref, deg_ref, row_ref, seq_ref, wl_ref, bl_ref,
              wr_ref, lw_ref, lb_ref, out_ref):
    deg = deg_ref[0, :, 0:1] + deg_ref[1, :, 0:1]
    recip = 1.0 / jnp.maximum(deg, 1.0)
    m0 = agg_ref[0] * recip
    m1 = agg_ref[1] * recip
    dn = (((1,), (1,)), ((), ()))
    wl = wl_ref[...]
    h = (lax.dot_general(m0, wl[:, :FH], dn, preferred_element_type=jnp.float32)
         + lax.dot_general(m1, wl[:, FH:], dn, preferred_element_type=jnp.float32)
         + bl_ref[...]
         + lax.dot_general(x_ref[...], wr_ref[...], dn,
                           preferred_element_type=jnp.float32))
    h = jnp.maximum(h, 0.0)

    rowv = row_ref[0, 0, :]
    onehot = (lax.broadcasted_iota(jnp.int32, (BN, L), 1)
              == rowv[:, None]).astype(jnp.float32)
    gate = lax.dot_general(onehot, seq_ref[...], (((1,), (0,)), ((), ())),
                           preferred_element_type=jnp.float32)
    h = h * gate

    logits = lax.dot_general(h, lw_ref[...], dn,
                             preferred_element_type=jnp.float32) + lb_ref[...]
    m = jnp.max(logits, axis=1, keepdims=True)
    e = jnp.exp(logits - m)
    out_ref[...] = (logits - m) - jnp.log(jnp.sum(e, axis=1, keepdims=True))


def kernel(x, edge_index, seq_reverse, seqid_reverse, row, ln_g, ln_b,
           llin_W, llin_b, rlin_W, rlin_b, gru_Wih, gru_Whh, gru_bih,
           gru_bhh, sage_Wl, sage_bl, sage_Wr, lin_W, lin_b):
    x2 = x.reshape(2 * N, FH)
    src_c = edge_index[0].reshape(NSUB, EPS)
    dst_c = edge_index[1].reshape(NSUB, EPS)
    body = jnp.concatenate([
        src_c[:, :NFB * BLK].reshape(NSUB, NFB, BLK),
        dst_c[:, :NFB * BLK].reshape(NSUB, NFB, BLK),
    ], axis=2).reshape(NSUB, NFB * 2 * BLK)
    tail = jnp.concatenate([src_c[:, NFB * BLK:], dst_c[:, NFB * BLK:]], axis=1)
    eint = jnp.concatenate([body, tail], axis=1).reshape(-1)

    agg, deg = _make_sc_segsum()(x2, eint)

    seq_out = pl.pallas_call(
        _seq_body,
        out_shape=jax.ShapeDtypeStruct((L, F), jnp.float32),
        scratch_shapes=[
            pltpu.VMEM((L, 3 * H), jnp.float32),
            pltpu.VMEM((L, H), jnp.float32),
        ],
    )(seq_reverse, ln_g.reshape(1, F), ln_b.reshape(1, F),
      llin_W, llin_b.reshape(1, H), rlin_W, rlin_b.reshape(1, H),
      gru_Wih, gru_Whh, gru_bih.reshape(1, 3 * H), gru_bhh.reshape(1, 3 * H))

    row3d = row.reshape(NBN, 1, BN)
    full = lambda *shape: pl.BlockSpec(shape, lambda i: (0,) * len(shape))
    out = pl.pallas_call(
        _fin_body,
        grid=(NBN,),
        in_specs=[
            pl.BlockSpec((BN, F), lambda i: (i, 0)),
            pl.BlockSpec((2, BN, FH), lambda i: (0, i, 0)),
            pl.BlockSpec((2, BN, FH), lambda i: (0, i, 0)),
            pl.BlockSpec((1, 1, BN), lambda i: (i, 0, 0)),
            full(L, F),
            full(H, F),
            full(1, H),
            full(H, F),
            full(C, H),
            full(1, C),
        ],
        out_specs=pl.BlockSpec((BN, C), lambda i: (i, 0)),
        out_shape=jax.ShapeDtypeStruct((N, C), jnp.float32),
    )(x, agg, deg, row3d, seq_out, sage_Wl, sage_bl.reshape(1, H),
      sage_Wr, lin_W, lin_b.reshape(1, C))
    return out
